# baseline (speedup 1.0000x reference)
import functools

import jax
import jax.numpy as jnp
from jax import lax
from jax.experimental import pallas as pl
from jax.experimental.pallas import tpu as pltpu
from jax.experimental.pallas import tpu_sc as plsc

SCALE_Q = 0.041666666666666664
DV_LATENT = 512
W_PACK = 384
D_UNPACK = 2 * W_PACK
N_CHUNKS_T = 8


@functools.lru_cache(maxsize=None)
def _make_sc_gather(S, T, K):
    info = plsc.get_sparse_core_info()
    nw = info.num_cores * info.num_subcores
    R = T * K
    assert R % nw == 0
    rows_per_w = R // nw
    chunk = 128
    assert rows_per_w % (2 * chunk) == 0 and K % chunk == 0
    n_pairs = rows_per_w // (2 * chunk)
    mesh = plsc.VectorSubcoreMesh(core_axis_name="c", subcore_axis_name="s")

    @functools.partial(
        pl.kernel,
        mesh=mesh,
        out_type=jax.ShapeDtypeStruct((R, W_PACK), jnp.int32),
        scratch_types=[
            pltpu.VMEM((1, rows_per_w), jnp.int32),
            pltpu.VMEM((chunk, W_PACK), jnp.int32),
            pltpu.VMEM((chunk, W_PACK), jnp.int32),
            pltpu.SemaphoreType.DMA,
            pltpu.SemaphoreType.DMA,
            pltpu.SemaphoreType.DMA,
            pltpu.SemaphoreType.DMA,
        ],
    )
    def gather_k(cache_hbm, idx_hbm, out_hbm, idx_v, rows_v0,
                 rows_v1, sem_g0, sem_g1, sem_w0, sem_w1):
        wid = lax.axis_index("s") * info.num_cores + lax.axis_index("c")
        base = wid * rows_per_w
        tok = base // K
        col = base % K
        pltpu.sync_copy(
            idx_hbm.at[pl.ds(tok, 1), pl.ds(col, rows_per_w)], idx_v
        )
        bufs = ((rows_v0, sem_w0), (rows_v1, sem_w1))

        def body(i, carry):
            pair0 = base + i * 2 * chunk

            @pl.when(i > 0)
            def _wait_prev():
                for b in range(2):
                    rows_v, sem_w = bufs[b]
                    pltpu.make_async_copy(
                        rows_v, out_hbm.at[pl.ds(base, chunk)], sem_w
                    ).wait()

            g0 = pltpu.async_copy(
                cache_hbm.at[idx_v.at[0, pl.ds(i * 2 * chunk, chunk)]],
                rows_v0, sem_g0,
            )
            g1 = pltpu.async_copy(
                cache_hbm.at[idx_v.at[0, pl.ds(i * 2 * chunk + chunk, chunk)]],
                rows_v1, sem_g1,
            )
            g0.wait()
            pltpu.async_copy(rows_v0, out_hbm.at[pl.ds(pair0, chunk)], sem_w0)
            g1.wait()
            pltpu.async_copy(
                rows_v1, out_hbm.at[pl.ds(pair0 + chunk, chunk)], sem_w1
            )
            return carry

        lax.fori_loop(0, n_pairs, body, 0)
        for b in range(2):
            rows_v, sem_w = bufs[b]
            pltpu.make_async_copy(
                rows_v, out_hbm.at[pl.ds(base, chunk)], sem_w
            ).wait()

    return gather_k


def _attn_body(q_ref, k_ref, sink_ref, o_ref):
    q = q_ref[0].astype(jnp.bfloat16)
    kw = k_ref[...]
    lo = lax.bitcast_convert_type(
        lax.shift_left(kw, 16), jnp.float32).astype(jnp.bfloat16)
    hi = lax.bitcast_convert_type(kw, jnp.float32).astype(jnp.bfloat16)
    s = sink_ref[...]
    logits = (
        lax.dot_general(q[:, :W_PACK], lo, (((1,), (1,)), ((), ())),
                        preferred_element_type=jnp.float32)
        + lax.dot_general(q[:, W_PACK:], hi, (((1,), (1,)), ((), ())),
                          preferred_element_type=jnp.float32)
    ) * SCALE_Q
    m = jnp.maximum(jnp.max(logits, axis=1, keepdims=True), s)
    p = jnp.exp(logits - m)
    denom = jnp.sum(p, axis=1, keepdims=True) + jnp.exp(s - m)
    attn = (p / denom).astype(jnp.bfloat16)
    out_lo = lax.dot_general(attn, lo, (((1,), (0,)), ((), ())),
                             preferred_element_type=jnp.float32)
    out_hi = lax.dot_general(attn, hi[:, :DV_LATENT - W_PACK],
                             (((1,), (0,)), ((), ())),
                             preferred_element_type=jnp.float32)
    o_ref[0] = jnp.concatenate([out_lo, out_hi], axis=1)


def _tc_attn(q, gathered, sink, interpret=False):
    T, H, D = q.shape
    K = gathered.shape[0] // T
    return pl.pallas_call(
        _attn_body,
        grid=(T,),
        in_specs=[
            pl.BlockSpec((1, H, D), lambda t: (t, 0, 0)),
            pl.BlockSpec((K, W_PACK), lambda t: (t, 0)),
            pl.BlockSpec((H, 1), lambda t: (0, 0)),
        ],
        out_specs=pl.BlockSpec((1, H, DV_LATENT), lambda t: (t, 0, 0)),
        out_shape=jax.ShapeDtypeStruct((T, H, DV_LATENT), jnp.float32),
        interpret=interpret,
    )(q, gathered, sink)


def kernel(q, kv_cache, topk_indices, attn_sink):
    T, H, D = q.shape
    K = topk_indices.shape[1]
    S = kv_cache.shape[0]
    kv16 = kv_cache.astype(jnp.bfloat16)
    kv768 = jnp.pad(kv16, ((0, 0), (0, 2 * W_PACK - D)))
    u_lo = lax.convert_element_type(
        lax.bitcast_convert_type(kv768[:, :W_PACK], jnp.uint16), jnp.uint32)
    u_hi = lax.convert_element_type(
        lax.bitcast_convert_type(kv768[:, W_PACK:], jnp.uint16), jnp.uint32)
    cache_w = lax.bitcast_convert_type(
        jnp.bitwise_or(u_lo, lax.shift_left(u_hi, jnp.uint32(16))), jnp.int32
    )
    q_p = jnp.pad(q, ((0, 0), (0, 0), (0, D_UNPACK - D)))
    sink = attn_sink.reshape(H, 1)
    tc = T // N_CHUNKS_T
    sizes = [tc] * (N_CHUNKS_T - 1) + [tc // 2, tc // 2]
    outs = []
    t0 = 0
    for sz in sizes:
        g = _make_sc_gather(S, sz, K)(cache_w, topk_indices[t0:t0 + sz])
        outs.append(_tc_attn(q_p[t0:t0 + sz], g, sink))
        t0 += sz
    return jnp.concatenate(outs, axis=0)


# speedup vs baseline: 1.0080x; 1.0080x over previous
import functools

import jax
import jax.numpy as jnp
from jax import lax
from jax.experimental import pallas as pl
from jax.experimental.pallas import tpu as pltpu
from jax.experimental.pallas import tpu_sc as plsc

SCALE_Q = 0.041666666666666664
DV_LATENT = 512
W_PACK = 384
D_UNPACK = 2 * W_PACK
N_CHUNKS_T = 8


@functools.lru_cache(maxsize=None)
def _make_sc_gather(S, T, K):
    info = plsc.get_sparse_core_info()
    nw = info.num_cores * info.num_subcores
    R = T * K
    assert R % nw == 0
    rows_per_w = R // nw
    chunk = 128
    assert rows_per_w % (2 * chunk) == 0 and K % chunk == 0
    n_pairs = rows_per_w // (2 * chunk)
    mesh = plsc.VectorSubcoreMesh(core_axis_name="c", subcore_axis_name="s")

    @functools.partial(
        pl.kernel,
        mesh=mesh,
        out_type=jax.ShapeDtypeStruct((R, W_PACK), jnp.int32),
        scratch_types=[
            pltpu.VMEM((1, rows_per_w), jnp.int32),
            pltpu.VMEM((chunk, W_PACK), jnp.int32),
            pltpu.VMEM((chunk, W_PACK), jnp.int32),
            pltpu.SemaphoreType.DMA,
            pltpu.SemaphoreType.DMA,
            pltpu.SemaphoreType.DMA,
            pltpu.SemaphoreType.DMA,
        ],
    )
    def gather_k(cache_hbm, idx_hbm, out_hbm, idx_v, rows_v0,
                 rows_v1, sem_g0, sem_g1, sem_w0, sem_w1):
        wid = lax.axis_index("s") * info.num_cores + lax.axis_index("c")
        base = wid * rows_per_w
        tok = base // K
        col = base % K
        pltpu.sync_copy(
            idx_hbm.at[pl.ds(tok, 1), pl.ds(col, rows_per_w)], idx_v
        )
        bufs = ((rows_v0, sem_w0), (rows_v1, sem_w1))

        def body(i, carry):
            pair0 = base + i * 2 * chunk

            @pl.when(i > 0)
            def _wait_prev():
                for b in range(2):
                    rows_v, sem_w = bufs[b]
                    pltpu.make_async_copy(
                        rows_v, out_hbm.at[pl.ds(base, chunk)], sem_w
                    ).wait()

            g0 = pltpu.async_copy(
                cache_hbm.at[idx_v.at[0, pl.ds(i * 2 * chunk, chunk)]],
                rows_v0, sem_g0,
            )
            g1 = pltpu.async_copy(
                cache_hbm.at[idx_v.at[0, pl.ds(i * 2 * chunk + chunk, chunk)]],
                rows_v1, sem_g1,
            )
            g0.wait()
            pltpu.async_copy(rows_v0, out_hbm.at[pl.ds(pair0, chunk)], sem_w0)
            g1.wait()
            pltpu.async_copy(
                rows_v1, out_hbm.at[pl.ds(pair0 + chunk, chunk)], sem_w1
            )
            return carry

        lax.fori_loop(0, n_pairs, body, 0)
        for b in range(2):
            rows_v, sem_w = bufs[b]
            pltpu.make_async_copy(
                rows_v, out_hbm.at[pl.ds(base, chunk)], sem_w
            ).wait()

    return gather_k


def _attn_body(q_ref, k_ref, sink_ref, o_ref):
    q = q_ref[0].astype(jnp.bfloat16)
    kw = k_ref[...]
    lo = lax.bitcast_convert_type(
        lax.shift_left(kw, 16), jnp.float32).astype(jnp.bfloat16)
    hi = lax.bitcast_convert_type(kw, jnp.float32).astype(jnp.bfloat16)
    s = sink_ref[...]
    logits = (
        lax.dot_general(q[:, :W_PACK], lo, (((1,), (1,)), ((), ())),
                        preferred_element_type=jnp.float32)
        + lax.dot_general(q[:, W_PACK:], hi, (((1,), (1,)), ((), ())),
                          preferred_element_type=jnp.float32)
    ) * SCALE_Q
    m = jnp.maximum(jnp.max(logits, axis=1, keepdims=True), s)
    p = jnp.exp(logits - m)
    denom = jnp.sum(p, axis=1, keepdims=True) + jnp.exp(s - m)
    attn = (p / denom).astype(jnp.bfloat16)
    out_lo = lax.dot_general(attn, lo, (((1,), (0,)), ((), ())),
                             preferred_element_type=jnp.float32)
    out_hi = lax.dot_general(attn, hi[:, :DV_LATENT - W_PACK],
                             (((1,), (0,)), ((), ())),
                             preferred_element_type=jnp.float32)
    o_ref[0] = jnp.concatenate([out_lo, out_hi], axis=1)


def _tc_attn(q, gathered, sink, interpret=False):
    T, H, D = q.shape
    K = gathered.shape[0] // T
    return pl.pallas_call(
        _attn_body,
        grid=(T,),
        in_specs=[
            pl.BlockSpec((1, H, D), lambda t: (t, 0, 0)),
            pl.BlockSpec((K, W_PACK), lambda t: (t, 0)),
            pl.BlockSpec((H, 1), lambda t: (0, 0)),
        ],
        out_specs=pl.BlockSpec((1, H, DV_LATENT), lambda t: (t, 0, 0)),
        out_shape=jax.ShapeDtypeStruct((T, H, DV_LATENT), jnp.float32),
        interpret=interpret,
    )(q, gathered, sink)


def kernel(q, kv_cache, topk_indices, attn_sink):
    T, H, D = q.shape
    K = topk_indices.shape[1]
    S = kv_cache.shape[0]
    kv16 = kv_cache.astype(jnp.bfloat16)
    kv768 = jnp.pad(kv16, ((0, 0), (0, 2 * W_PACK - D)))
    u_lo = lax.convert_element_type(
        lax.bitcast_convert_type(kv768[:, :W_PACK], jnp.uint16), jnp.uint32)
    u_hi = lax.convert_element_type(
        lax.bitcast_convert_type(kv768[:, W_PACK:], jnp.uint16), jnp.uint32)
    cache_w = lax.bitcast_convert_type(
        jnp.bitwise_or(u_lo, lax.shift_left(u_hi, jnp.uint32(16))), jnp.int32
    )
    q_p = jnp.pad(q, ((0, 0), (0, 0), (0, D_UNPACK - D)))
    sink = attn_sink.reshape(H, 1)
    tc = T // N_CHUNKS_T
    gather = _make_sc_gather(S, tc, K)
    outs = []
    for c in range(N_CHUNKS_T):
        g = gather(cache_w, topk_indices[c * tc:(c + 1) * tc])
        outs.append(_tc_attn(q_p[c * tc:(c + 1) * tc], g, sink))
    return jnp.concatenate(outs, axis=0)
